# baseline (speedup 1.0000x reference)
import math

import jax
import jax.numpy as jnp
from jax import lax
from jax.experimental import pallas as pl
from jax.experimental.pallas import tpu as pltpu

S = 64.0
M = 0.5
COS_M = math.cos(M)
SIN_M = math.sin(M)

B = 2048
C = 20000
WM = 19968
WT = C - WM
WP = 20096
BM = 128
NBUF = 2
NR = B // BM


def _body(logits_hbm, tail_ref, lab_ref, diffp_hbm, st_ref, stm_ref,
          inm, outm, semim, semom):
    def in_copy(r, slot):
        return pltpu.make_async_copy(
            logits_hbm.at[pl.ds(r * BM, BM), pl.ds(0, WM)],
            inm.at[slot], semim.at[slot])

    def out_copy(r, slot):
        return pltpu.make_async_copy(
            outm.at[slot], diffp_hbm.at[pl.ds(r * BM, BM)], semom.at[slot])

    for i in range(NBUF):
        in_copy(i, i).start()

    def step(r, carry):
        slot = lax.rem(r, NBUF)

        @pl.when(r >= NBUF)
        def _wait_out_slot():
            out_copy(r - NBUF, slot).wait()

        in_copy(r, slot).wait()

        xm = inm[slot]
        xt = tail_ref[pl.ds(r * BM, BM), :]
        lab = lab_ref[pl.ds(r * BM, BM), :]

        cols_m = lax.broadcasted_iota(jnp.int32, (BM, WM), 1)
        cols_t = lax.broadcasted_iota(jnp.int32, (BM, WT), 1) + WM
        t = (jnp.sum(jnp.where(cols_m == lab, xm, 0.0), axis=1, keepdims=True)
             + jnp.sum(jnp.where(cols_t == lab, xt, 0.0), axis=1, keepdims=True))
        sin_t = jnp.sqrt(jnp.maximum(1.0 - t * t, 0.0))
        final = t * COS_M - sin_t * SIN_M
        st_ref[pl.ds(r * BM, BM), :] = sin_t
        stm_ref[pl.ds(r * BM, BM), :] = sin_t * COS_M + t * SIN_M
        tgt2 = final * S

        hi_m = jnp.concatenate([xm[:, 1:], xt[:, :1]], axis=1)
        outm[slot, :, pl.ds(0, WM)] = jnp.where(cols_m >= lab, hi_m, xm) * S - tgt2
        pad = jnp.zeros((BM, WP - WM - WT), jnp.float32)
        lo_t = jnp.concatenate([xt, pad], axis=1)
        hi_t = jnp.concatenate([xt[:, 1:], pad, pad[:, :1]], axis=1)
        tcols = lax.broadcasted_iota(jnp.int32, (BM, WP - WM), 1) + WM
        outm[slot, :, pl.ds(WM, WP - WM)] = (
            jnp.where(tcols >= lab, hi_t, lo_t) * S - tgt2)

        out_copy(r, slot).start()

        @pl.when(r + NBUF < NR)
        def _start_next_in():
            in_copy(r + NBUF, slot).start()

        return carry

    lax.fori_loop(0, NR, step, None)

    for i in range(NBUF):
        r = NR - NBUF + i
        out_copy(r, r % NBUF).wait()


def kernel(logits, labels):
    b, c = logits.shape
    lab2 = labels.reshape(b, 1)
    tail = lax.slice(logits, (0, WM), (b, c))
    diffp, st, stm = pl.pallas_call(
        _body,
        in_specs=[
            pl.BlockSpec(memory_space=pltpu.MemorySpace.HBM),
            pl.BlockSpec(memory_space=pltpu.MemorySpace.VMEM),
            pl.BlockSpec(memory_space=pltpu.MemorySpace.VMEM),
        ],
        out_specs=[
            pl.BlockSpec(memory_space=pltpu.MemorySpace.HBM),
            pl.BlockSpec(memory_space=pltpu.MemorySpace.VMEM),
            pl.BlockSpec(memory_space=pltpu.MemorySpace.VMEM),
        ],
        out_shape=[
            jax.ShapeDtypeStruct((b, WP), jnp.float32),
            jax.ShapeDtypeStruct((b, 1), jnp.float32),
            jax.ShapeDtypeStruct((b, 1), jnp.float32),
        ],
        scratch_shapes=[
            pltpu.VMEM((NBUF, BM, WM), jnp.float32),
            pltpu.VMEM((NBUF, BM, WP), jnp.float32),
            pltpu.SemaphoreType.DMA((NBUF,)),
            pltpu.SemaphoreType.DMA((NBUF,)),
        ],
    )(logits, tail, lab2)
    diff = lax.slice(diffp, (0, 0), (b, c - 1))
    sin_m = jnp.full((b,), math.sin(M), dtype=logits.dtype)
    return diff, st.reshape(b), stm.reshape(b), sin_m


# speedup vs baseline: 1.0113x; 1.0113x over previous
import math

import jax
import jax.numpy as jnp
from jax import lax
from jax.experimental import pallas as pl
from jax.experimental.pallas import tpu as pltpu

S = 64.0
M = 0.5
COS_M = math.cos(M)
SIN_M = math.sin(M)

B = 2048
C = 20000
WM = 19968
WT = C - WM
WP = 20096
BM = 64
NBUF = 4
NR = B // BM


def _body(logits_hbm, tail_ref, lab_ref, diffp_hbm, st_ref, stm_ref,
          inm, outm, semim, semom):
    def in_copy(r, slot):
        return pltpu.make_async_copy(
            logits_hbm.at[pl.ds(r * BM, BM), pl.ds(0, WM)],
            inm.at[slot], semim.at[slot])

    def out_copy(r, slot):
        return pltpu.make_async_copy(
            outm.at[slot], diffp_hbm.at[pl.ds(r * BM, BM)], semom.at[slot])

    for i in range(NBUF):
        in_copy(i, i).start()

    def step(r, carry):
        slot = lax.rem(r, NBUF)

        @pl.when(r >= NBUF)
        def _wait_out_slot():
            out_copy(r - NBUF, slot).wait()

        in_copy(r, slot).wait()

        xm = inm[slot]
        xt = tail_ref[pl.ds(r * BM, BM), :]
        lab = lab_ref[pl.ds(r * BM, BM), :]

        cols_m = lax.broadcasted_iota(jnp.int32, (BM, WM), 1)
        cols_t = lax.broadcasted_iota(jnp.int32, (BM, WT), 1) + WM
        t = (jnp.sum(jnp.where(cols_m == lab, xm, 0.0), axis=1, keepdims=True)
             + jnp.sum(jnp.where(cols_t == lab, xt, 0.0), axis=1, keepdims=True))
        sin_t = jnp.sqrt(jnp.maximum(1.0 - t * t, 0.0))
        final = t * COS_M - sin_t * SIN_M
        st_ref[pl.ds(r * BM, BM), :] = sin_t
        stm_ref[pl.ds(r * BM, BM), :] = sin_t * COS_M + t * SIN_M
        tgt2 = final * S

        hi_m = jnp.concatenate([xm[:, 1:], xt[:, :1]], axis=1)
        outm[slot, :, pl.ds(0, WM)] = jnp.where(cols_m >= lab, hi_m, xm) * S - tgt2
        pad = jnp.zeros((BM, WP - WM - WT), jnp.float32)
        lo_t = jnp.concatenate([xt, pad], axis=1)
        hi_t = jnp.concatenate([xt[:, 1:], pad, pad[:, :1]], axis=1)
        tcols = lax.broadcasted_iota(jnp.int32, (BM, WP - WM), 1) + WM
        outm[slot, :, pl.ds(WM, WP - WM)] = (
            jnp.where(tcols >= lab, hi_t, lo_t) * S - tgt2)

        out_copy(r, slot).start()

        @pl.when(r + NBUF < NR)
        def _start_next_in():
            in_copy(r + NBUF, slot).start()

        return carry

    lax.fori_loop(0, NR, step, None)

    for i in range(NBUF):
        r = NR - NBUF + i
        out_copy(r, r % NBUF).wait()


def kernel(logits, labels):
    b, c = logits.shape
    lab2 = labels.reshape(b, 1)
    tail = lax.slice(logits, (0, WM), (b, c))
    diffp, st, stm = pl.pallas_call(
        _body,
        in_specs=[
            pl.BlockSpec(memory_space=pltpu.MemorySpace.HBM),
            pl.BlockSpec(memory_space=pltpu.MemorySpace.VMEM),
            pl.BlockSpec(memory_space=pltpu.MemorySpace.VMEM),
        ],
        out_specs=[
            pl.BlockSpec(memory_space=pltpu.MemorySpace.HBM),
            pl.BlockSpec(memory_space=pltpu.MemorySpace.VMEM),
            pl.BlockSpec(memory_space=pltpu.MemorySpace.VMEM),
        ],
        out_shape=[
            jax.ShapeDtypeStruct((b, WP), jnp.float32),
            jax.ShapeDtypeStruct((b, 1), jnp.float32),
            jax.ShapeDtypeStruct((b, 1), jnp.float32),
        ],
        scratch_shapes=[
            pltpu.VMEM((NBUF, BM, WM), jnp.float32),
            pltpu.VMEM((NBUF, BM, WP), jnp.float32),
            pltpu.SemaphoreType.DMA((NBUF,)),
            pltpu.SemaphoreType.DMA((NBUF,)),
        ],
    )(logits, tail, lab2)
    diff = lax.slice(diffp, (0, 0), (b, c - 1))
    sin_m = jnp.full((b,), math.sin(M), dtype=logits.dtype)
    return diff, st.reshape(b), stm.reshape(b), sin_m
